# baseline (speedup 1.0000x reference)
import functools

import jax
import jax.numpy as jnp
from jax import lax
from jax.experimental import pallas as pl
from jax.experimental.pallas import tpu as pltpu
from jax.experimental.pallas import tpu_sc as plsc

F = 26
V = 100000
D = 16
B = 4096

NC = 2
NS = 16
NW = NC * NS
ROWS = F * B
RPW = ROWS // NW
CH = 128
NCH = RPW // CH

NCOL = 2048
NB = NCOL // D
NSTEP = (B * D) // NCOL


def _gather_body(tbl_hbm, idx_hbm, out_hbm, idx_v, rows_v, sem):
    wid = lax.axis_index("s") * NC + lax.axis_index("c")
    pltpu.sync_copy(idx_hbm.at[wid], idx_v)
    copies = []
    for j in range(NCH):
        copies.append(
            pltpu.async_copy(tbl_hbm.at[idx_v.at[j]],
                             rows_v.at[pl.ds(j * CH, CH)], sem))
    for c in copies:
        c.wait()
    pltpu.sync_copy(rows_v, out_hbm.at[wid])


@functools.lru_cache(maxsize=1)
def _gather():
    return functools.partial(
        pl.kernel,
        mesh=plsc.VectorSubcoreMesh(core_axis_name="c", subcore_axis_name="s"),
        compiler_params=pltpu.CompilerParams(use_tc_tiling_on_sc=False),
        out_type=jax.ShapeDtypeStruct((NW, RPW, D), jnp.float32),
        scratch_types=[
            pltpu.VMEM((NCH, CH), jnp.int32),
            pltpu.VMEM((RPW, D), jnp.float32),
            pltpu.SemaphoreType.DMA,
        ],
    )(_gather_body)


def _fwfm_body(e_ref, a_ref, wt_ref, r_ref, o_ref):
    e = e_ref[...]
    p = jnp.dot(a_ref[...], e, preferred_element_type=jnp.float32)
    y = e * (p + wt_ref[...])
    colsum = jnp.sum(y, axis=0, keepdims=True)
    o = jnp.dot(colsum, r_ref[...], preferred_element_type=jnp.float32)
    o_ref[...] = jax.nn.sigmoid(o).reshape(1, 1, NB)


def kernel(x, tables, field_cov, fwfm_linear_w):
    tbl = tables.reshape(F * V, D)
    idx = (x.T.astype(jnp.int32)
           + (jnp.arange(F, dtype=jnp.int32) * V)[:, None])
    idx = idx.reshape(NW, NCH, CH)

    emb = _gather()(tbl, idx)
    e2 = emb.reshape(F, B * D)

    sym = (field_cov + field_cov.T) * 0.5
    a_half = 0.5 * (sym - jnp.diag(jnp.diag(sym)))
    wtile = jnp.tile(fwfm_linear_w, (1, NCOL // D))
    col = jnp.arange(NCOL, dtype=jnp.int32) // D
    r_sel = (col[:, None] == jnp.arange(NB, dtype=jnp.int32)[None, :]
             ).astype(jnp.float32)

    out2 = pl.pallas_call(
        _fwfm_body,
        grid=(NSTEP,),
        in_specs=[
            pl.BlockSpec((F, NCOL), lambda c: (0, c)),
            pl.BlockSpec((F, F), lambda c: (0, 0)),
            pl.BlockSpec((F, NCOL), lambda c: (0, 0)),
            pl.BlockSpec((NCOL, NB), lambda c: (0, 0)),
        ],
        out_specs=pl.BlockSpec((1, 1, NB), lambda c: (c, 0, 0)),
        out_shape=jax.ShapeDtypeStruct((NSTEP, 1, NB), jnp.float32),
    )(e2, a_half, wtile, r_sel)
    return out2.reshape(B)


# speedup vs baseline: 3.1459x; 3.1459x over previous
import functools

import jax
import jax.numpy as jnp
from jax import lax
from jax.experimental import pallas as pl
from jax.experimental.pallas import tpu as pltpu
from jax.experimental.pallas import tpu_sc as plsc

F = 26
V = 100000
D = 16
B = 4096

NC = 2
NS = 16
NW = NC * NS
R = F * D

NCOL = 1024
NSTEP = B // NCOL


def _gather_body(tbl_hbm, idx_hbm, out_hbm, idx_v, rows_v, sem):
    wid = lax.axis_index("s") * NC + lax.axis_index("c")

    @pl.when(wid < F)
    def _():
        f = wid
        pltpu.sync_copy(idx_hbm.at[f], idx_v)
        copies = []
        for s in range(D):
            copies.append(
                pltpu.async_copy(tbl_hbm.at[f * D + s].at[idx_v],
                                 rows_v.at[s], sem))
        for c in copies:
            c.wait()
        pltpu.sync_copy(rows_v, out_hbm.at[f])


@functools.lru_cache(maxsize=1)
def _gather():
    return functools.partial(
        pl.kernel,
        mesh=plsc.VectorSubcoreMesh(core_axis_name="c", subcore_axis_name="s"),
        compiler_params=pltpu.CompilerParams(use_tc_tiling_on_sc=False),
        out_type=jax.ShapeDtypeStruct((F, D, B), jnp.float32),
        scratch_types=[
            pltpu.VMEM((B,), jnp.int32),
            pltpu.VMEM((D, B), jnp.float32),
            pltpu.SemaphoreType.DMA,
        ],
    )(_gather_body)


def _fwfm_body(e_ref, k_ref, w_ref, o_ref):
    e = e_ref[...]
    p = jnp.dot(k_ref[...], e, preferred_element_type=jnp.float32)
    y = e * (p + w_ref[...])
    s = jnp.sum(y, axis=0, keepdims=True)
    o_ref[...] = jax.nn.sigmoid(s)


def kernel(x, tables, field_cov, fwfm_linear_w):
    m = jnp.transpose(tables, (0, 2, 1)).reshape(R, V)
    xt = x.T.astype(jnp.int32)

    et = _gather()(m, xt).reshape(R, B)

    sym = (field_cov + field_cov.T) * 0.5
    a_half = 0.5 * (sym - jnp.diag(jnp.diag(sym)))
    kr = jnp.kron(a_half, jnp.eye(D, dtype=jnp.float32))
    w416 = fwfm_linear_w.reshape(R, 1)

    out2 = pl.pallas_call(
        _fwfm_body,
        grid=(NSTEP,),
        in_specs=[
            pl.BlockSpec((R, NCOL), lambda c: (0, c)),
            pl.BlockSpec((R, R), lambda c: (0, 0)),
            pl.BlockSpec((R, 1), lambda c: (0, 0)),
        ],
        out_specs=pl.BlockSpec((1, NCOL), lambda c: (0, c)),
        out_shape=jax.ShapeDtypeStruct((1, B), jnp.float32),
    )(et, kr, w416)
    return out2.reshape(B)
